# baseline (speedup 1.0000x reference)
import functools
import math

import jax
import jax.numpy as jnp
import numpy as np
from jax import lax
from jax.experimental import pallas as pl
from jax.experimental.pallas import tpu as pltpu
from jax.experimental.pallas import tpu_sc as plsc

VOCAB = 100000
D_MODEL = 128
LENGTH = 2048
BATCH = 4
SEQ = 2048
SCALE = math.sqrt(float(D_MODEL))

_INFO = plsc.get_sparse_core_info()
_NC = _INFO.num_cores
_NS = _INFO.num_subcores
_NW = _NC * _NS
_PPW = SEQ // _NW
_LANES = 16
_HALF = D_MODEL // 2


def _pos_factors():
    rate = 1.0 / (10000.0 ** (np.arange(_HALF, dtype=np.float64) / _HALF))
    d_ang = np.arange(_PPW, dtype=np.float64)[:, None] * rate[None, :]
    dtab = np.concatenate([np.cos(d_ang), np.sin(d_ang)], axis=-1)
    w_ang = (np.arange(_NW, dtype=np.float64) * _PPW)[:, None] * rate[None, :]
    wtab = np.concatenate([np.sin(w_ang), np.cos(w_ang)], axis=-1)
    return (
        jnp.asarray(dtab.astype(np.float32)),
        jnp.asarray(wtab.reshape(-1).astype(np.float32)),
    )


_DTAB, _WTAB = _pos_factors()


@functools.partial(
    pl.kernel,
    mesh=plsc.VectorSubcoreMesh(core_axis_name="c", subcore_axis_name="s"),
    out_type=jax.ShapeDtypeStruct((BATCH, SEQ, D_MODEL), jnp.float32),
    scratch_types=[
        pltpu.VMEM((BATCH, _PPW), jnp.int32),
        pltpu.VMEM((BATCH, _PPW, D_MODEL), jnp.float32),
        pltpu.VMEM((_PPW, D_MODEL), jnp.float32),
        pltpu.VMEM((D_MODEL,), jnp.float32),
        pltpu.SemaphoreType.DMA,
        pltpu.SemaphoreType.DMA,
        pltpu.SemaphoreType.DMA,
        pltpu.SemaphoreType.DMA,
        pltpu.SemaphoreType.DMA,
        pltpu.SemaphoreType.DMA,
        pltpu.SemaphoreType.DMA,
    ],
)
def _emb_kernel(x_hbm, table_hbm, dtab_hbm, wtab_hbm, out_hbm,
                idx_v, rows_v, dtab_v, wrow_v,
                sem_idx, sem_pos, sem_out, sg0, sg1, sg2, sg3):
    wid = lax.axis_index("s") * _NC + lax.axis_index("c")
    gsems = [sg0, sg1, sg2, sg3]
    idx_cps = [
        pltpu.async_copy(x_hbm.at[b, pl.ds(wid * _PPW, _PPW)], idx_v.at[b], sem_idx)
        for b in range(BATCH)
    ]
    for cp in idx_cps:
        cp.wait()
    gcps = [pltpu.async_copy(table_hbm.at[idx_v.at[0]], rows_v.at[0], gsems[0])]
    dtab_cp = pltpu.async_copy(dtab_hbm, dtab_v, sem_pos)
    wrow_cp = pltpu.async_copy(
        wtab_hbm.at[pl.ds(wid * D_MODEL, D_MODEL)], wrow_v, sem_pos
    )
    for b in range(1, BATCH):
        gcps.append(
            pltpu.async_copy(table_hbm.at[idx_v.at[b]], rows_v.at[b], gsems[b])
        )
    dtab_cp.wait()
    wrow_cp.wait()

    sw = [wrow_v[pl.ds(c * _LANES, _LANES)] for c in range(_HALF // _LANES)]
    cw = [wrow_v[pl.ds(_HALF + c * _LANES, _LANES)] for c in range(_HALF // _LANES)]

    def make_body(b):
        def row_body(i, _):
            for c in range(_HALF // _LANES):
                cd = dtab_v[i, pl.ds(c * _LANES, _LANES)]
                sd = dtab_v[i, pl.ds(_HALF + c * _LANES, _LANES)]
                sla = pl.ds(c * _LANES, _LANES)
                slb = pl.ds(_HALF + c * _LANES, _LANES)
                rows_v[b, i, sla] = rows_v[b, i, sla] * SCALE + (
                    sw[c] * cd + cw[c] * sd
                )
                rows_v[b, i, slb] = rows_v[b, i, slb] * SCALE + (
                    cw[c] * cd - sw[c] * sd
                )
            return _
        return row_body

    out_cps = []
    for b in range(BATCH):
        gcps[b].wait()
        lax.fori_loop(0, _PPW, make_body(b), 0)
        out_cps.append(
            pltpu.async_copy(
                rows_v.at[b],
                out_hbm.at[b, pl.ds(wid * _PPW, _PPW)],
                sem_out,
            )
        )
    for cp in out_cps:
        cp.wait()


def kernel(x, table):
    return _emb_kernel(x.astype(jnp.int32), table, _DTAB, _WTAB)


# speedup vs baseline: 1.0227x; 1.0227x over previous
import functools
import math

import jax
import jax.numpy as jnp
import numpy as np
from jax import lax
from jax.experimental import pallas as pl
from jax.experimental.pallas import tpu as pltpu
from jax.experimental.pallas import tpu_sc as plsc

VOCAB = 100000
D_MODEL = 128
LENGTH = 2048
BATCH = 4
SEQ = 2048
SCALE = math.sqrt(float(D_MODEL))

_INFO = plsc.get_sparse_core_info()
_NC = _INFO.num_cores
_NS = _INFO.num_subcores
_NW = _NC * _NS
_PPW = SEQ // _NW
_LANES = 16
_VPR = D_MODEL // _LANES
_HALF = D_MODEL // 2
_FLEN = _PPW * D_MODEL + _NW * D_MODEL


def _pos_factors():
    rate = 1.0 / (10000.0 ** (np.arange(_HALF, dtype=np.float64) / _HALF))
    d_ang = np.arange(_PPW, dtype=np.float64)[:, None] * rate[None, :]
    dtab = np.concatenate([np.cos(d_ang), np.sin(d_ang)], axis=-1)
    w_ang = (np.arange(_NW, dtype=np.float64) * _PPW)[:, None] * rate[None, :]
    wtab = np.concatenate([np.sin(w_ang), np.cos(w_ang)], axis=-1)
    flat = np.concatenate([dtab.reshape(-1), wtab.reshape(-1)])
    return jnp.asarray(flat.astype(np.float32))


_FACTORS = _pos_factors()


@functools.partial(
    pl.kernel,
    mesh=plsc.VectorSubcoreMesh(core_axis_name="c", subcore_axis_name="s"),
    out_type=jax.ShapeDtypeStruct((BATCH, SEQ, D_MODEL), jnp.float32),
    scratch_types=[
        pltpu.VMEM((BATCH, _PPW), jnp.int32),
        pltpu.VMEM((BATCH, _PPW, D_MODEL), jnp.float32),
        pltpu.VMEM((_PPW * D_MODEL,), jnp.float32),
        pltpu.VMEM((_PPW, D_MODEL), jnp.float32),
        pltpu.VMEM((D_MODEL,), jnp.float32),
        pltpu.SemaphoreType.DMA,
        pltpu.SemaphoreType.DMA,
        pltpu.SemaphoreType.DMA,
        pltpu.SemaphoreType.DMA,
        pltpu.SemaphoreType.DMA,
        pltpu.SemaphoreType.DMA,
        pltpu.SemaphoreType.DMA,
    ],
)
def _emb_kernel(x_hbm, table_hbm, fac_hbm, out_hbm,
                idx_v, rows_v, dtab_v, pos_v, wrow_v,
                sem_idx, sem_pos, sem_out, sg0, sg1, sg2, sg3):
    wid = lax.axis_index("s") * _NC + lax.axis_index("c")
    gsems = [sg0, sg1, sg2, sg3]
    idx_cps = [
        pltpu.async_copy(x_hbm.at[b, pl.ds(wid * _PPW, _PPW)], idx_v.at[b], sem_idx)
        for b in range(BATCH)
    ]
    idx_cps[0].wait()
    gcps = [pltpu.async_copy(table_hbm.at[idx_v.at[0]], rows_v.at[0], gsems[0])]
    dtab_cp = pltpu.async_copy(
        fac_hbm.at[pl.ds(0, _PPW * D_MODEL)], dtab_v, sem_pos
    )
    wrow_cp = pltpu.async_copy(
        fac_hbm.at[pl.ds(_PPW * D_MODEL + wid * D_MODEL, D_MODEL)], wrow_v, sem_pos
    )
    for b in range(1, BATCH):
        idx_cps[b].wait()
        gcps.append(
            pltpu.async_copy(table_hbm.at[idx_v.at[b]], rows_v.at[b], gsems[b])
        )
    dtab_cp.wait()
    wrow_cp.wait()

    sw = [wrow_v[pl.ds(c * _LANES, _LANES)] for c in range(_HALF // _LANES)]
    cw = [wrow_v[pl.ds(_HALF + c * _LANES, _LANES)] for c in range(_HALF // _LANES)]

    def pos_body(i, _):
        for c in range(_HALF // _LANES):
            cd = dtab_v[pl.ds(i * D_MODEL + c * _LANES, _LANES)]
            sd = dtab_v[pl.ds(i * D_MODEL + _HALF + c * _LANES, _LANES)]
            pos_v[i, pl.ds(c * _LANES, _LANES)] = sw[c] * cd + cw[c] * sd
            pos_v[i, pl.ds(_HALF + c * _LANES, _LANES)] = cw[c] * cd - sw[c] * sd
        return _

    lax.fori_loop(0, _PPW, pos_body, 0)

    def make_body(b):
        def row_body(i, _):
            for c in range(_VPR):
                sl = pl.ds(c * _LANES, _LANES)
                rows_v[b, i, sl] = rows_v[b, i, sl] * SCALE + pos_v[i, sl]
            return _
        return row_body

    out_cps = []
    for b in range(BATCH):
        gcps[b].wait()
        lax.fori_loop(0, _PPW, make_body(b), 0)
        out_cps.append(
            pltpu.async_copy(
                rows_v.at[b],
                out_hbm.at[b, pl.ds(wid * _PPW, _PPW)],
                sem_out,
            )
        )
    for cp in out_cps:
        cp.wait()


def kernel(x, table):
    return _emb_kernel(x.astype(jnp.int32), table, _FACTORS)
